# Initial kernel scaffold; baseline (speedup 1.0000x reference)
import jax
import jax.numpy as jnp
from jax.experimental import pallas as pl


def kernel(x, batch_graph, W_att, w1, w2, w3, w_end):
    raise NotImplementedError("write your pallas kernel here")


# speedup vs baseline: 16.0866x; 16.0866x over previous
import functools

import jax
import jax.numpy as jnp
from jax.experimental import pallas as pl

N = 4096
KNN = 12
R = 256
_NEG = -1e30


def _leaky(v):
    return jnp.where(v >= 0, v, 0.2 * v)


def _prep_body(bgp_ref, wnb_ref, wd_ref, f_ref, sq_ref, at_ref, bct_ref):
    bgp = bgp_ref[...]
    f = bgp.T
    f_ref[...] = f
    sq_ref[...] = jnp.sum(f * f, axis=0, keepdims=True)
    at_ref[...] = jnp.dot(bgp, wnb_ref[...], preferred_element_type=jnp.float32)
    bct_ref[...] = jnp.dot(bgp, wd_ref[...], preferred_element_type=jnp.float32)


def _knn_body(ft_ref, f_ref, sq_ref, at_ref, bct_ref, u_ref, s_ref, q_ref):
    i = pl.program_id(0)
    fb = ft_ref[...]
    d = 2.0 * jnp.dot(fb, f_ref[...], preferred_element_type=jnp.float32) \
        - sq_ref[...]
    at = at_ref[...]
    o_ch = at.shape[1]
    iota = jax.lax.broadcasted_iota(jnp.int32, (R, N), 1)
    s = jnp.zeros((R, o_ch), jnp.float32)
    q = jnp.zeros((R, o_ch), jnp.float32)
    mx = jnp.full((R, o_ch), _NEG, jnp.float32)
    for _ in range(KNN):
        m = jnp.max(d, axis=1, keepdims=True)
        eq = d >= m
        idxv = jnp.min(jnp.where(eq, iota, N), axis=1, keepdims=True)
        oh = iota == idxv
        g = jnp.dot(oh.astype(jnp.float32), at,
                    preferred_element_type=jnp.float32)
        s = s + g
        q = q + g * g
        mx = jnp.maximum(mx, g)
        d = jnp.where(oh, _NEG, d)
    bct = bct_ref[...]
    u_ref[...] = mx + bct
    sblk = jnp.sum(s + KNN * bct, axis=0, keepdims=True)
    qblk = jnp.sum(q + 2.0 * bct * s + KNN * bct * bct, axis=0, keepdims=True)

    @pl.when(i == 0)
    def _():
        s_ref[...] = jnp.zeros_like(s_ref)
        q_ref[...] = jnp.zeros_like(q_ref)

    s_ref[...] += sblk
    q_ref[...] += qblk


def _norm(u, s, q):
    cnt = jnp.float32(N * KNN)
    mean = s / cnt
    var = q / cnt - mean * mean
    return _leaky((u - mean) * jax.lax.rsqrt(var + 1e-5))


def _fin_body(u_ref, s_ref, q_ref, wnb_ref, wd_ref,
              ft2_ref, f2_ref, sq2_ref, at2_ref, bct2_ref):
    xt = _norm(u_ref[...], s_ref[...], q_ref[...])
    ft2_ref[...] = xt
    f2 = xt.T
    f2_ref[...] = f2
    sq2_ref[...] = jnp.sum(f2 * f2, axis=0, keepdims=True)
    at2_ref[...] = jnp.dot(xt, wnb_ref[...], preferred_element_type=jnp.float32)
    bct2_ref[...] = jnp.dot(xt, wd_ref[...], preferred_element_type=jnp.float32)


def _final_body(u_ref, s_ref, q_ref, wend_ref, watt_ref, out_ref):
    x3 = _norm(u_ref[...], s_ref[...], q_ref[...])
    a = jnp.dot(x3, wend_ref[...], preferred_element_type=jnp.float32)
    m2 = jnp.mean(a, axis=0, keepdims=True)
    v2 = jnp.mean(a * a, axis=0, keepdims=True) - m2 * m2
    ab = _leaky((a - m2) * jax.lax.rsqrt(v2 + 1e-5))
    gc = jnp.mean(jnp.dot(ab, watt_ref[...], preferred_element_type=jnp.float32),
                  axis=0, keepdims=True)
    tg = jnp.tanh(gc)
    logit = jnp.sum(ab * tg, axis=1, keepdims=True)
    scores = 1.0 / (1.0 + jnp.exp(-logit))
    out_ref[...] = jnp.sum(ab * scores, axis=0, keepdims=True)


def _make_knn(c, o):
    f32 = jnp.float32
    return pl.pallas_call(
        _knn_body,
        grid=(N // R,),
        in_specs=[
            pl.BlockSpec((R, c), lambda i: (i, 0)),
            pl.BlockSpec((c, N), lambda i: (0, 0)),
            pl.BlockSpec((1, N), lambda i: (0, 0)),
            pl.BlockSpec((N, o), lambda i: (0, 0)),
            pl.BlockSpec((R, o), lambda i: (i, 0)),
        ],
        out_specs=[
            pl.BlockSpec((R, o), lambda i: (i, 0)),
            pl.BlockSpec((1, o), lambda i: (0, 0)),
            pl.BlockSpec((1, o), lambda i: (0, 0)),
        ],
        out_shape=[
            jax.ShapeDtypeStruct((N, o), f32),
            jax.ShapeDtypeStruct((1, o), f32),
            jax.ShapeDtypeStruct((1, o), f32),
        ],
    )


def kernel(x, batch_graph, W_att, w1, w2, w3, w_end):
    f32 = jnp.float32
    bgp = jnp.pad(batch_graph.astype(f32), ((0, 0), (0, 5)))
    w1nb = jnp.pad(w1[:, :3].T, ((0, 5), (0, 0)))
    w1d = jnp.pad((w1[:, 3:] - w1[:, :3]).T, ((0, 5), (0, 0)))
    w2nbT = w2[:, :64].T
    w2dT = (w2[:, 64:] - w2[:, :64]).T
    w3nbT = w3[:, :64].T
    w3dT = (w3[:, 64:] - w3[:, :64]).T

    prep = pl.pallas_call(
        _prep_body,
        out_shape=[
            jax.ShapeDtypeStruct((8, N), f32),
            jax.ShapeDtypeStruct((1, N), f32),
            jax.ShapeDtypeStruct((N, 64), f32),
            jax.ShapeDtypeStruct((N, 64), f32),
        ],
    )
    f1, sq1, a1t, bct1 = prep(bgp, w1nb, w1d)

    u1, s1, q1 = _make_knn(8, 64)(bgp, f1, sq1, a1t, bct1)

    fin1 = pl.pallas_call(
        _fin_body,
        out_shape=[
            jax.ShapeDtypeStruct((N, 64), f32),
            jax.ShapeDtypeStruct((64, N), f32),
            jax.ShapeDtypeStruct((1, N), f32),
            jax.ShapeDtypeStruct((N, 64), f32),
            jax.ShapeDtypeStruct((N, 64), f32),
        ],
    )
    ft2, f2, sq2, a2t, bct2 = fin1(u1, s1, q1, w2nbT, w2dT)

    u2, s2, q2 = _make_knn(64, 64)(ft2, f2, sq2, a2t, bct2)

    fin2 = pl.pallas_call(
        _fin_body,
        out_shape=[
            jax.ShapeDtypeStruct((N, 64), f32),
            jax.ShapeDtypeStruct((64, N), f32),
            jax.ShapeDtypeStruct((1, N), f32),
            jax.ShapeDtypeStruct((N, 32), f32),
            jax.ShapeDtypeStruct((N, 32), f32),
        ],
    )
    ft3, f3, sq3, a3t, bct3 = fin2(u2, s2, q2, w3nbT, w3dT)

    u3, s3, q3 = _make_knn(64, 32)(ft3, f3, sq3, a3t, bct3)

    final = pl.pallas_call(
        _final_body,
        out_shape=jax.ShapeDtypeStruct((1, 32), f32),
    )
    pooled = final(u3, s3, q3, w_end.T, W_att).reshape(32)

    gf = jnp.concatenate([x, jnp.zeros((x.shape[0], 32), x.dtype)], axis=1)
    return (gf, pooled)


# speedup vs baseline: 18.8087x; 1.1692x over previous
import functools

import jax
import jax.numpy as jnp
from jax import lax
from jax.experimental import pallas as pl
from jax.experimental.pallas import tpu as pltpu
from jax.experimental.pallas import tpu_sc as plsc

N = 4096
KNN = 12
R = 256
NW = 32
P = N // NW
_NEG = -1e30


def _leaky(v):
    return jnp.where(v >= 0, v, 0.2 * v)


def _prep_body(bgp_ref, wnb_ref, wd_ref, f_ref, sq_ref, at_ref, bct_ref):
    bgp = bgp_ref[...]
    f = bgp.T
    f_ref[...] = f
    sq_ref[...] = jnp.sum(f * f, axis=0, keepdims=True)
    at_ref[...] = jnp.dot(bgp, wnb_ref[...], preferred_element_type=jnp.float32)
    bct_ref[...] = jnp.dot(bgp, wd_ref[...], preferred_element_type=jnp.float32)


def _knn_body(ft_ref, f_ref, sq_ref, idx_ref):
    i = pl.program_id(0)
    fb = ft_ref[...]
    d = 2.0 * jnp.dot(fb, f_ref[...], preferred_element_type=jnp.float32) \
        - sq_ref[...]
    iota = jax.lax.broadcasted_iota(jnp.int32, (R, N), 1)
    for t in range(KNN):
        m = jnp.max(d, axis=1, keepdims=True)
        eq = d >= m
        idxv = jnp.min(jnp.where(eq, iota, N), axis=1, keepdims=True)
        idx_ref[:, pl.ds(t, 1)] = idxv
        d = jnp.where(eq, _NEG, d)
    own = jax.lax.broadcasted_iota(jnp.int32, (R, 4), 0) + i * R
    idx_ref[:, pl.ds(KNN, 4)] = own


def _make_knn(c):
    return pl.pallas_call(
        _knn_body,
        grid=(N // R,),
        in_specs=[
            pl.BlockSpec((R, c), lambda i: (i, 0)),
            pl.BlockSpec((c, N), lambda i: (0, 0)),
            pl.BlockSpec((1, N), lambda i: (0, 0)),
        ],
        out_specs=pl.BlockSpec((R, 16), lambda i: (i, 0)),
        out_shape=jax.ShapeDtypeStruct((N, 16), jnp.int32),
    )


def _make_scgather(o, interpret=False):
    f32 = jnp.float32
    mesh = plsc.VectorSubcoreMesh(core_axis_name="c", subcore_axis_name="s",
                                  num_cores=2, num_subcores=16)

    @functools.partial(
        pl.kernel,
        out_type=[
            jax.ShapeDtypeStruct((N, o), f32),
            jax.ShapeDtypeStruct((NW, o), f32),
            jax.ShapeDtypeStruct((NW, o), f32),
        ],
        mesh=mesh,
        scratch_types=[
            pltpu.VMEM((P * 16,), jnp.int32),
            pltpu.VMEM((256, 128), f32),
            pltpu.VMEM((256, 128), f32),
            pltpu.VMEM((P, o), f32),
            pltpu.VMEM((P, o), f32),
            pltpu.VMEM((o,), f32),
            pltpu.VMEM((o,), f32),
            pltpu.SemaphoreType.DMA,
            pltpu.SemaphoreType.DMA,
        ],
        interpret=interpret,
    )
    def k(at_hbm, idxf_hbm, bct_hbm, u_hbm, sp_hbm, qp_hbm,
          idx_v, rows_a, rows_b, bct_v, u_v, sacc_v, qacc_v, sem_a, sem_b):
        wid = lax.axis_index("s") * 2 + lax.axis_index("c")
        base = wid * P
        pltpu.sync_copy(idxf_hbm.at[pl.ds(base * 16, P * 16)], idx_v)
        pltpu.sync_copy(bct_hbm.at[pl.ds(base, P)], bct_v)
        for ov in range(o // 16):
            sl = pl.ds(ov * 16, 16)
            sacc_v[sl] = jnp.zeros((16,), f32)
            qacc_v[sl] = jnp.zeros((16,), f32)
        bufs = (rows_a, rows_b)
        sems = (sem_a, sem_b)
        nr = 8
        cps = {}

        def fire(r):
            buf, sem = bufs[r % 2], sems[r % 2]
            for g in range(2):
                cps[(r, g)] = pltpu.async_copy(
                    at_hbm.at[idx_v.at[pl.ds(r * 256 + g * 128, 128)]],
                    buf.at[pl.ds(g * 128, 128)], sem)

        fire(0)
        for r in range(nr):
            buf = bufs[r % 2]
            for g in range(2):
                cps[(r, g)].wait()
            if r + 1 < nr:
                fire(r + 1)

            def body(p, carry):
                pt = r * 16 + p
                for ov in range(o // 16):
                    sl = pl.ds(ov * 16, 16)
                    b = bct_v[pt, sl]
                    s = jnp.zeros((16,), f32)
                    q = jnp.zeros((16,), f32)
                    m = jnp.full((16,), _NEG, f32)
                    for t in range(KNN):
                        v = buf[p * 16 + t, sl]
                        s = s + v
                        q = q + v * v
                        m = jnp.maximum(m, v)
                    u_v[pt, sl] = m + b
                    sacc_v[sl] = sacc_v[sl] + s + float(KNN) * b
                    qacc_v[sl] = qacc_v[sl] + q + 2.0 * b * s + float(KNN) * b * b
                return carry

            lax.fori_loop(0, 16, body, 0)
        pltpu.sync_copy(u_v, u_hbm.at[pl.ds(base, P)])
        pltpu.sync_copy(sacc_v, sp_hbm.at[wid])
        pltpu.sync_copy(qacc_v, qp_hbm.at[wid])

    return k


def _norm(u, sp, qp):
    cnt = jnp.float32(N * KNN)
    mean = jnp.sum(sp, axis=0, keepdims=True) / cnt
    var = jnp.sum(qp, axis=0, keepdims=True) / cnt - mean * mean
    return _leaky((u - mean) * jax.lax.rsqrt(var + 1e-5))


def _fin_body(u_ref, sp_ref, qp_ref, wnb_ref, wd_ref,
              ft2_ref, f2_ref, sq2_ref, at2_ref, bct2_ref):
    xt = _norm(u_ref[...], sp_ref[...], qp_ref[...])
    ft2_ref[...] = xt
    f2 = xt.T
    f2_ref[...] = f2
    sq2_ref[...] = jnp.sum(f2 * f2, axis=0, keepdims=True)
    at2_ref[...] = jnp.dot(xt, wnb_ref[...], preferred_element_type=jnp.float32)
    bct2_ref[...] = jnp.dot(xt, wd_ref[...], preferred_element_type=jnp.float32)


def _final_body(u_ref, sp_ref, qp_ref, wend_ref, watt_ref, out_ref):
    x3 = _norm(u_ref[...], sp_ref[...], qp_ref[...])
    a = jnp.dot(x3, wend_ref[...], preferred_element_type=jnp.float32)
    m2 = jnp.mean(a, axis=0, keepdims=True)
    v2 = jnp.mean(a * a, axis=0, keepdims=True) - m2 * m2
    ab = _leaky((a - m2) * jax.lax.rsqrt(v2 + 1e-5))
    gc = jnp.mean(jnp.dot(ab, watt_ref[...], preferred_element_type=jnp.float32),
                  axis=0, keepdims=True)
    tg = jnp.tanh(gc)
    logit = jnp.sum(ab * tg, axis=1, keepdims=True)
    scores = 1.0 / (1.0 + jnp.exp(-logit))
    out_ref[...] = jnp.sum(ab * scores, axis=0, keepdims=True)


SC_INTERPRET = False


def kernel(x, batch_graph, W_att, w1, w2, w3, w_end):
    f32 = jnp.float32
    bgp = jnp.pad(batch_graph.astype(f32), ((0, 0), (0, 5)))
    w1nb = jnp.pad(w1[:, :3].T, ((0, 5), (0, 64)))
    w1d = jnp.pad((w1[:, 3:] - w1[:, :3]).T, ((0, 5), (0, 0)))
    w2nbT = jnp.pad(w2[:, :64].T, ((0, 0), (0, 64)))
    w2dT = (w2[:, 64:] - w2[:, :64]).T
    w3nbT = jnp.pad(w3[:, :64].T, ((0, 0), (0, 96)))
    w3dT = (w3[:, 64:] - w3[:, :64]).T

    prep = pl.pallas_call(
        _prep_body,
        out_shape=[
            jax.ShapeDtypeStruct((8, N), f32),
            jax.ShapeDtypeStruct((1, N), f32),
            jax.ShapeDtypeStruct((N, 128), f32),
            jax.ShapeDtypeStruct((N, 64), f32),
        ],
    )
    f1, sq1, a1t, bct1 = prep(bgp, w1nb, w1d)

    idx1 = _make_knn(8)(bgp, f1, sq1)
    u1, sp1, qp1 = _make_scgather(64, SC_INTERPRET)(
        a1t, idx1.reshape(N * 16), bct1)

    fin1 = pl.pallas_call(
        _fin_body,
        out_shape=[
            jax.ShapeDtypeStruct((N, 64), f32),
            jax.ShapeDtypeStruct((64, N), f32),
            jax.ShapeDtypeStruct((1, N), f32),
            jax.ShapeDtypeStruct((N, 128), f32),
            jax.ShapeDtypeStruct((N, 64), f32),
        ],
    )
    ft2, f2, sq2, a2t, bct2 = fin1(u1, sp1, qp1, w2nbT, w2dT)

    idx2 = _make_knn(64)(ft2, f2, sq2)
    u2, sp2, qp2 = _make_scgather(64, SC_INTERPRET)(
        a2t, idx2.reshape(N * 16), bct2)

    fin2 = pl.pallas_call(
        _fin_body,
        out_shape=[
            jax.ShapeDtypeStruct((N, 64), f32),
            jax.ShapeDtypeStruct((64, N), f32),
            jax.ShapeDtypeStruct((1, N), f32),
            jax.ShapeDtypeStruct((N, 128), f32),
            jax.ShapeDtypeStruct((N, 32), f32),
        ],
    )
    ft3, f3, sq3, a3t, bct3 = fin2(u2, sp2, qp2, w3nbT, w3dT)

    idx3 = _make_knn(64)(ft3, f3, sq3)
    u3, sp3, qp3 = _make_scgather(32, SC_INTERPRET)(
        a3t, idx3.reshape(N * 16), bct3)

    final = pl.pallas_call(
        _final_body,
        out_shape=jax.ShapeDtypeStruct((1, 32), f32),
    )
    pooled = final(u3, sp3, qp3, w_end.T, W_att).reshape(32)

    gf = jnp.concatenate([x, jnp.zeros((x.shape[0], 32), x.dtype)], axis=1)
    return (gf, pooled)
